# baseline (speedup 1.0000x reference)
import jax
import jax.numpy as jnp
import numpy as np
from jax.experimental import pallas as pl
from jax.experimental.pallas import tpu as pltpu
from jax.experimental.pallas import tpu_sc as plsc

H = 128
R = 50
B = 64
K = 16
CUTOFF = 5.0


def _pick_block(n, prefs):
    for p in prefs:
        if n % p == 0:
            return p
    return n


_LANES = 128
_IMAX = 2147483647


def _nbr_body(pos_blk, cx2, cy2, cz2, brow_blk, bcol2,
              src_ref, dist_ref, valid_ref):
    i = pl.program_id(0)
    bn = pos_blk.shape[0]
    na = cx2.shape[0]
    px = pos_blk[:, 0:1].reshape(bn, 1, 1)
    py = pos_blk[:, 1:2].reshape(bn, 1, 1)
    pz = pos_blk[:, 2:3].reshape(bn, 1, 1)
    dx = px - cx2[...][None, :, :]
    dy = py - cy2[...][None, :, :]
    dz = pz - cz2[...][None, :, :]
    d2 = dx * dx + dy * dy + dz * dz
    b_r = brow_blk[:, 0:1].reshape(bn, 1, 1)
    a3 = jax.lax.broadcasted_iota(jnp.int32, (bn, na, _LANES), 1)
    l3 = jax.lax.broadcasted_iota(jnp.int32, (bn, na, _LANES), 2)
    col3 = a3 * _LANES + l3
    row_g = i * bn + jax.lax.broadcasted_iota(jnp.int32, (bn, 1), 0)
    bad = ((b_r != bcol2[...][None, :, :]) | (col3 == row_g.reshape(bn, 1, 1))
           | (d2 >= CUTOFF * CUTOFF))
    kb = jax.lax.bitcast_convert_type(d2, jnp.int32)
    key3 = jnp.where(bad, _IMAX, (kb & (-128)) | a3)
    lane2 = jax.lax.broadcasted_iota(jnp.int32, (bn, _LANES), 1)
    t = jnp.full((bn, _LANES), -_IMAX - 1, jnp.int32)
    idx_cols, dist_cols, valid_cols = [], [], []
    for _ in range(K):
        m = jnp.min(jnp.where(key3 > t[:, None, :], key3, _IMAX), axis=1)
        rowmin = jnp.min(m, axis=1, keepdims=True)
        lsel = jnp.min(jnp.where(m == rowmin, lane2, _LANES),
                       axis=1, keepdims=True)
        t = jnp.where(lane2 == lsel, rowmin, t)
        v = rowmin != _IMAX
        a_k = rowmin & 127
        col_k = a_k * _LANES + lsel
        qd2 = jax.lax.bitcast_convert_type(rowmin & (-128), jnp.float32)
        idx_cols.append(jnp.where(v, col_k, row_g))
        dist_cols.append(jnp.where(v, jnp.sqrt(qd2), 0.0))
        valid_cols.append(v.astype(jnp.float32))
    src_ref[...] = jnp.concatenate(idx_cols, axis=1)
    dist_ref[...] = jnp.concatenate(dist_cols, axis=1)
    valid_ref[...] = jnp.concatenate(valid_cols, axis=1)


def _neighbors_pallas(pos, batch):
    n = pos.shape[0]
    bn = _pick_block(n, [80, 40, 8])
    npad = ((n + _LANES - 1) // _LANES) * _LANES
    na = npad // _LANES
    posT = jnp.zeros((3, npad), jnp.float32).at[:, :n].set(pos.T)
    cx2 = posT[0].reshape(na, _LANES)
    cy2 = posT[1].reshape(na, _LANES)
    cz2 = posT[2].reshape(na, _LANES)
    bcol = jnp.full((npad,), -1, jnp.int32).at[:n].set(batch.astype(jnp.int32))
    bcol2 = bcol.reshape(na, _LANES)
    b2d = batch.reshape(n, 1).astype(jnp.int32)
    out = pl.pallas_call(
        _nbr_body,
        grid=(n // bn,),
        in_specs=[
            pl.BlockSpec((bn, 3), lambda i: (i, 0)),
            pl.BlockSpec((na, _LANES), lambda i: (0, 0)),
            pl.BlockSpec((na, _LANES), lambda i: (0, 0)),
            pl.BlockSpec((na, _LANES), lambda i: (0, 0)),
            pl.BlockSpec((bn, 1), lambda i: (i, 0)),
            pl.BlockSpec((na, _LANES), lambda i: (0, 0)),
        ],
        out_specs=[
            pl.BlockSpec((bn, K), lambda i: (i, 0)),
            pl.BlockSpec((bn, K), lambda i: (i, 0)),
            pl.BlockSpec((bn, K), lambda i: (i, 0)),
        ],
        out_shape=[
            jax.ShapeDtypeStruct((n, K), jnp.int32),
            jax.ShapeDtypeStruct((n, K), jnp.float32),
            jax.ShapeDtypeStruct((n, K), jnp.float32),
        ],
    )(pos, cx2, cy2, cz2, b2d, bcol2)
    return out


def _embed_body(an_blk, table, out_ref):
    an = an_blk[:, 0:1]
    nv = table.shape[0]
    io = jax.lax.broadcasted_iota(jnp.int32, (1, nv), 1)
    oh = (an == io).astype(jnp.float32)
    out_ref[...] = jnp.dot(oh, table[...], preferred_element_type=jnp.float32)


def _embed_pallas(atomic_numbers, table):
    n = atomic_numbers.shape[0]
    bn = _pick_block(n, [400, 200, 40, 8])
    an2 = atomic_numbers.reshape(n, 1).astype(jnp.int32)
    return pl.pallas_call(
        _embed_body,
        grid=(n // bn,),
        in_specs=[
            pl.BlockSpec((bn, 1), lambda i: (i, 0)),
            pl.BlockSpec(table.shape, lambda i: (0, 0)),
        ],
        out_specs=pl.BlockSpec((bn, H), lambda i: (i, 0)),
        out_shape=jax.ShapeDtypeStruct((n, H), jnp.float32),
    )(an2, table)


def _gather_rows(x, idx_flat):
    e0 = idx_flat.shape[0]
    w = 256
    unit = w * 32
    e = ((e0 + unit - 1) // unit) * unit
    if e != e0:
        idx_flat = jnp.concatenate(
            [idx_flat, jnp.zeros((e - e0,), idx_flat.dtype)])
    idx2 = idx_flat.reshape(1, e)
    mesh = plsc.VectorSubcoreMesh(core_axis_name="c", subcore_axis_name="s")

    @pl.kernel(
        out_type=jax.ShapeDtypeStruct((e, x.shape[1]), x.dtype), mesh=mesh
    )
    def k(x_hbm, i_hbm, o_hbm):
        def body(i_vmem, o_vmem):
            pltpu.sync_copy(x_hbm.at[i_vmem.at[0]], o_vmem)

        pltpu.emit_pipeline(
            body,
            grid=(e // w,),
            in_specs=[pl.BlockSpec((1, w), index_map=lambda i: (0, i))],
            out_specs=[
                pl.BlockSpec((w, x.shape[1]), index_map=lambda i: (i, 0))
            ],
            core_axis_name=("c", "s"),
            dimension_semantics=(pltpu.PARALLEL,),
        )(i_hbm, o_hbm)

    return k(x, idx2)[:e0]


def _silu(x):
    return x * jax.nn.sigmoid(x)


def _conv_body(x_blk, xj_blk, de_blk, ve_blk, widths, centers,
               ee1, bee1, ee2, bee2, gw, gb, m1, bm1, m2, bm2,
               u1, bu1, u2, bu2, lng, lnb, out_ref):
    bn = x_blk.shape[0]
    bnk = xj_blk.shape[0]
    d1 = de_blk[...]
    wdt = widths[...]
    gamma = 1.0 / (2.0 * wdt * wdt)
    rbf = jnp.exp(-gamma * (d1 - centers[...]) ** 2)
    cut = (0.5 * (jnp.cos(np.float32(np.pi) * d1 / CUTOFF) + 1.0)
           * (d1 < CUTOFF).astype(jnp.float32))
    ea = rbf * cut

    def mm(a, b):
        return jnp.dot(a, b, preferred_element_type=jnp.float32)

    def mmb(a, b):
        return jnp.dot(a.astype(jnp.bfloat16), b.astype(jnp.bfloat16),
                       preferred_element_type=jnp.float32)

    h1 = _silu(mm(ea, ee1[...]) + bee1[...])
    ee = mmb(h1, ee2[...]) + bee2[...]
    gate = jax.nn.sigmoid(mmb(ee, gw[...]) + gb[...])
    x = x_blk[...]
    m1w = m1[...]
    pre_i = mm(x, m1w[0:H])
    pre_j = mmb(xj_blk[...], m1w[H:2 * H])
    pre_e = mmb(ee, m1w[2 * H:3 * H]) + bm1[...]
    pre = (pre_j + pre_e).reshape(bn, K, H) + pre_i[:, None, :]
    msg = mmb(_silu(pre).reshape(bnk, H), m2[...]) + bm2[...]
    m = msg * gate * ve_blk[...]
    aggr = m.reshape(bn, K, H).sum(axis=1)
    u1w = u1[...]
    uh = _silu(mm(aggr, u1w[0:H]) + mm(x, u1w[H:2 * H]) + bu1[...])
    upd = mm(uh, u2[...]) + bu2[...]
    xn = x + upd
    mean = xn.mean(axis=1, keepdims=True)
    var = ((xn - mean) ** 2).mean(axis=1, keepdims=True)
    out_ref[...] = (xn - mean) / jnp.sqrt(var + 1e-5) * lng[...] + lnb[...]


def _conv_pallas(x, xj, de, ve, c, centers):
    n = x.shape[0]
    bn = _pick_block(n, [400, 200, 40, 8])
    wts = [
        c['widths2d'], centers,
        c['ee1'][0], c['ee1'][1], c['ee2'][0], c['ee2'][1],
        c['g'][0], c['g'][1],
        c['m1'][0], c['m1'][1], c['m2'][0], c['m2'][1],
        c['u1'][0], c['u1'][1], c['u2'][0], c['u2'][1],
        c['ln_g'], c['ln_b'],
    ]
    in_specs = [
        pl.BlockSpec((bn, H), lambda i: (i, 0)),
        pl.BlockSpec((bn * K, H), lambda i: (i, 0)),
        pl.BlockSpec((bn * K, 1), lambda i: (i, 0)),
        pl.BlockSpec((bn * K, 1), lambda i: (i, 0)),
    ] + [pl.BlockSpec(w.shape, lambda i: (0, 0)) for w in wts]
    return pl.pallas_call(
        _conv_body,
        grid=(n // bn,),
        in_specs=in_specs,
        out_specs=pl.BlockSpec((bn, H), lambda i: (i, 0)),
        out_shape=jax.ShapeDtypeStruct((n, H), jnp.float32),
    )(x, xj, de, ve, *wts)


def _readout_body(x2, x3, x4, b2d, ro1, bro1, ro2, bro2,
                  gn1, bgn1, gn2, bgn2, pn, bpn,
                  af_ref, ae_ref, gf_ref):
    def mm(a, b):
        return jnp.dot(a, b, preferred_element_type=jnp.float32)

    af = (x2[...] + x3[...] + x4[...]) / 3.0
    ae = mm(_silu(mm(af, ro1[...]) + bro1[...]), ro2[...]) + bro2[...]
    g1 = jnp.tanh(mm(ae, gn1[...]) + bgn1[...])
    g = mm(g1, gn2[...]) + bgn2[...]
    bvec = b2d[...]
    io64 = jax.lax.broadcasted_iota(jnp.int32, (1, B), 1)
    ohb = bvec == io64
    ohf = ohb.astype(jnp.float32)
    neg = jnp.float32(-1e38)
    gmax = jnp.max(jnp.where(ohb, g, neg), axis=0, keepdims=True)
    gmax = jnp.where(gmax < -1e37, 0.0, gmax)
    gmax_atom = jax.lax.dot_general(
        ohf, gmax, (((1,), (1,)), ((), ())),
        preferred_element_type=jnp.float32)
    ge = jnp.exp(g - gmax_atom)
    gs = jnp.sum(ohf * ge, axis=0, keepdims=True)
    gs_atom = jax.lax.dot_general(
        ohf, gs, (((1,), (1,)), ((), ())),
        preferred_element_type=jnp.float32)
    alpha = ge / gs_atom
    h = _silu(mm(ae, pn[...]) + bpn[...])
    gf = jax.lax.dot_general(
        ohf, alpha * h, (((0,), (0,)), ((), ())),
        preferred_element_type=jnp.float32)
    af_ref[...] = af
    ae_ref[...] = ae
    gf_ref[...] = gf


def _readout_pallas(x2, x3, x4, batch, params):
    n = x2.shape[0]
    b2d = batch.reshape(n, 1).astype(jnp.int32)
    wts = [
        params['ro1'][0], params['ro1'][1].reshape(1, H),
        params['ro2'][0], params['ro2'][1].reshape(1, H),
        params['gn1'][0], params['gn1'][1].reshape(1, H),
        params['gn2'][0], params['gn2'][1].reshape(1, 1),
        params['pn'][0], params['pn'][1].reshape(1, H),
    ]
    return pl.pallas_call(
        _readout_body,
        out_shape=[
            jax.ShapeDtypeStruct((n, H), jnp.float32),
            jax.ShapeDtypeStruct((n, H), jnp.float32),
            jax.ShapeDtypeStruct((B, H), jnp.float32),
        ],
    )(x2, x3, x4, b2d, *wts)


def kernel(atomic_numbers, pos, batch, params):
    n = pos.shape[0]
    pos = pos.astype(jnp.float32)
    src, dist, valid = _neighbors_pallas(pos, batch)
    x = _embed_pallas(atomic_numbers, params['embed'])
    e = n * K
    src_flat = src.reshape(e)
    de = dist.reshape(e, 1)
    ve = valid.reshape(e, 1)
    centers = jnp.linspace(0.0, CUTOFF, R).reshape(1, R).astype(jnp.float32)
    feats = [x]
    for c in params['convs']:
        cc = dict(c)
        cc['widths2d'] = params['widths'].reshape(1, R)
        cc['ee1'] = (c['ee1'][0], c['ee1'][1].reshape(1, H))
        cc['ee2'] = (c['ee2'][0], c['ee2'][1].reshape(1, H))
        cc['g'] = (c['g'][0], c['g'][1].reshape(1, H))
        cc['m1'] = (c['m1'][0], c['m1'][1].reshape(1, H))
        cc['m2'] = (c['m2'][0], c['m2'][1].reshape(1, H))
        cc['u1'] = (c['u1'][0], c['u1'][1].reshape(1, H))
        cc['u2'] = (c['u2'][0], c['u2'][1].reshape(1, H))
        cc['ln_g'] = c['ln_g'].reshape(1, H)
        cc['ln_b'] = c['ln_b'].reshape(1, H)
        half = n // 2
        he = half * K
        xj1 = _gather_rows(x, src_flat[:he])
        xj2 = _gather_rows(x, src_flat[he:])
        x1 = _conv_pallas(x[:half], xj1, de[:he], ve[:he], cc, centers)
        x2 = _conv_pallas(x[half:], xj2, de[he:], ve[he:], cc, centers)
        x = jnp.concatenate([x1, x2], axis=0)
        feats.append(x)
    af, ae, gf = _readout_pallas(feats[-3], feats[-2], feats[-1], batch, params)
    return af, ae, gf


# speedup vs baseline: 1.0260x; 1.0260x over previous
import jax
import jax.numpy as jnp
import numpy as np
from jax.experimental import pallas as pl
from jax.experimental.pallas import tpu as pltpu
from jax.experimental.pallas import tpu_sc as plsc

H = 128
R = 50
B = 64
K = 16
CUTOFF = 5.0


def _pick_block(n, prefs):
    for p in prefs:
        if n % p == 0:
            return p
    return n


_LANES = 128
_IMAX = 2147483647


def _nbr_body(pos_blk, cx2, cy2, cz2, brow_blk, bcol2,
              src_ref, dist_ref, valid_ref):
    i = pl.program_id(0)
    bn = pos_blk.shape[0]
    na = cx2.shape[0]
    px = pos_blk[:, 0:1].reshape(bn, 1, 1)
    py = pos_blk[:, 1:2].reshape(bn, 1, 1)
    pz = pos_blk[:, 2:3].reshape(bn, 1, 1)
    dx = px - cx2[...][None, :, :]
    dy = py - cy2[...][None, :, :]
    dz = pz - cz2[...][None, :, :]
    d2 = dx * dx + dy * dy + dz * dz
    b_r = brow_blk[:, 0:1].reshape(bn, 1, 1)
    a3 = jax.lax.broadcasted_iota(jnp.int32, (bn, na, _LANES), 1)
    l3 = jax.lax.broadcasted_iota(jnp.int32, (bn, na, _LANES), 2)
    col3 = a3 * _LANES + l3
    row_g = i * bn + jax.lax.broadcasted_iota(jnp.int32, (bn, 1), 0)
    bad = ((b_r != bcol2[...][None, :, :]) | (col3 == row_g.reshape(bn, 1, 1))
           | (d2 >= CUTOFF * CUTOFF))
    kb = jax.lax.bitcast_convert_type(d2, jnp.int32)
    key3 = jnp.where(bad, _IMAX, (kb & (-128)) | a3)
    lane2 = jax.lax.broadcasted_iota(jnp.int32, (bn, _LANES), 1)
    t = jnp.full((bn, _LANES), -_IMAX - 1, jnp.int32)
    idx_cols, dist_cols, valid_cols = [], [], []
    for _ in range(K):
        m = jnp.min(jnp.where(key3 > t[:, None, :], key3, _IMAX), axis=1)
        rowmin = jnp.min(m, axis=1, keepdims=True)
        lsel = jnp.min(jnp.where(m == rowmin, lane2, _LANES),
                       axis=1, keepdims=True)
        t = jnp.where(lane2 == lsel, rowmin, t)
        v = rowmin != _IMAX
        a_k = rowmin & 127
        col_k = a_k * _LANES + lsel
        qd2 = jax.lax.bitcast_convert_type(rowmin & (-128), jnp.float32)
        idx_cols.append(jnp.where(v, col_k, row_g))
        dist_cols.append(jnp.where(v, jnp.sqrt(qd2), 0.0))
        valid_cols.append(v.astype(jnp.float32))
    src_ref[...] = jnp.concatenate(idx_cols, axis=1)
    dist_ref[...] = jnp.concatenate(dist_cols, axis=1)
    valid_ref[...] = jnp.concatenate(valid_cols, axis=1)


def _neighbors_pallas(pos, batch):
    n = pos.shape[0]
    bn = _pick_block(n, [80, 40, 8])
    npad = ((n + _LANES - 1) // _LANES) * _LANES
    na = npad // _LANES
    posT = jnp.zeros((3, npad), jnp.float32).at[:, :n].set(pos.T)
    cx2 = posT[0].reshape(na, _LANES)
    cy2 = posT[1].reshape(na, _LANES)
    cz2 = posT[2].reshape(na, _LANES)
    bcol = jnp.full((npad,), -1, jnp.int32).at[:n].set(batch.astype(jnp.int32))
    bcol2 = bcol.reshape(na, _LANES)
    b2d = batch.reshape(n, 1).astype(jnp.int32)
    out = pl.pallas_call(
        _nbr_body,
        grid=(n // bn,),
        in_specs=[
            pl.BlockSpec((bn, 3), lambda i: (i, 0)),
            pl.BlockSpec((na, _LANES), lambda i: (0, 0)),
            pl.BlockSpec((na, _LANES), lambda i: (0, 0)),
            pl.BlockSpec((na, _LANES), lambda i: (0, 0)),
            pl.BlockSpec((bn, 1), lambda i: (i, 0)),
            pl.BlockSpec((na, _LANES), lambda i: (0, 0)),
        ],
        out_specs=[
            pl.BlockSpec((bn, K), lambda i: (i, 0)),
            pl.BlockSpec((bn, K), lambda i: (i, 0)),
            pl.BlockSpec((bn, K), lambda i: (i, 0)),
        ],
        out_shape=[
            jax.ShapeDtypeStruct((n, K), jnp.int32),
            jax.ShapeDtypeStruct((n, K), jnp.float32),
            jax.ShapeDtypeStruct((n, K), jnp.float32),
        ],
    )(pos, cx2, cy2, cz2, b2d, bcol2)
    return out


def _embed_body(an_blk, table, out_ref):
    an = an_blk[:, 0:1]
    nv = table.shape[0]
    io = jax.lax.broadcasted_iota(jnp.int32, (1, nv), 1)
    oh = (an == io).astype(jnp.float32)
    out_ref[...] = jnp.dot(oh, table[...], preferred_element_type=jnp.float32)


def _embed_pallas(atomic_numbers, table):
    n = atomic_numbers.shape[0]
    bn = _pick_block(n, [400, 200, 40, 8])
    an2 = atomic_numbers.reshape(n, 1).astype(jnp.int32)
    return pl.pallas_call(
        _embed_body,
        grid=(n // bn,),
        in_specs=[
            pl.BlockSpec((bn, 1), lambda i: (i, 0)),
            pl.BlockSpec(table.shape, lambda i: (0, 0)),
        ],
        out_specs=pl.BlockSpec((bn, H), lambda i: (i, 0)),
        out_shape=jax.ShapeDtypeStruct((n, H), jnp.float32),
    )(an2, table)


def _gather_rows(x, idx_flat):
    e0 = idx_flat.shape[0]
    w = 256
    unit = w * 32
    e = ((e0 + unit - 1) // unit) * unit
    if e != e0:
        idx_flat = jnp.concatenate(
            [idx_flat, jnp.zeros((e - e0,), idx_flat.dtype)])
    idx2 = idx_flat.reshape(1, e)
    mesh = plsc.VectorSubcoreMesh(core_axis_name="c", subcore_axis_name="s")

    @pl.kernel(
        out_type=jax.ShapeDtypeStruct((e, x.shape[1]), x.dtype), mesh=mesh
    )
    def k(x_hbm, i_hbm, o_hbm):
        def body(i_vmem, o_vmem):
            pltpu.sync_copy(x_hbm.at[i_vmem.at[0]], o_vmem)

        pltpu.emit_pipeline(
            body,
            grid=(e // w,),
            in_specs=[pl.BlockSpec((1, w), index_map=lambda i: (0, i))],
            out_specs=[
                pl.BlockSpec((w, x.shape[1]), index_map=lambda i: (i, 0))
            ],
            core_axis_name=("c", "s"),
            dimension_semantics=(pltpu.PARALLEL,),
        )(i_hbm, o_hbm)

    return k(x, idx2)[:e0]


def _silu(x):
    return x * jax.nn.sigmoid(x)


def _conv_body(x_blk, xj_blk, de_blk, ve_blk, widths, centers,
               ee1, bee1, ee2, bee2, gw, gb, m1, bm1, m2, bm2,
               u1, bu1, u2, bu2, lng, lnb, out_ref):
    bn = x_blk.shape[0]
    bnk = xj_blk.shape[0]
    d1 = de_blk[...]
    wdt = widths[...]
    gamma = 1.0 / (2.0 * wdt * wdt)
    rbf = jnp.exp(-gamma * (d1 - centers[...]) ** 2)
    cut = (0.5 * (jnp.cos(np.float32(np.pi) * d1 / CUTOFF) + 1.0)
           * (d1 < CUTOFF).astype(jnp.float32))
    ea = rbf * cut

    def mm(a, b):
        return jnp.dot(a, b, preferred_element_type=jnp.float32)

    h1 = _silu(mm(ea, ee1[...]) + bee1[...])
    ee = mm(h1, ee2[...]) + bee2[...]
    gate = jax.nn.sigmoid(mm(ee, gw[...]) + gb[...])
    x = x_blk[...]
    m1w = m1[...]
    pre_i = mm(x, m1w[0:H])
    pre_j = mm(xj_blk[...].astype(jnp.float32), m1w[H:2 * H])
    pre_e = mm(ee, m1w[2 * H:3 * H]) + bm1[...]
    pre = (pre_j + pre_e).reshape(bn, K, H) + pre_i[:, None, :]
    msg = mm(_silu(pre).reshape(bnk, H), m2[...]) + bm2[...]
    m = msg * gate * ve_blk[...]
    aggr = m.reshape(bn, K, H).sum(axis=1)
    u1w = u1[...]
    uh = _silu(mm(aggr, u1w[0:H]) + mm(x, u1w[H:2 * H]) + bu1[...])
    upd = mm(uh, u2[...]) + bu2[...]
    xn = x + upd
    mean = xn.mean(axis=1, keepdims=True)
    var = ((xn - mean) ** 2).mean(axis=1, keepdims=True)
    out_ref[...] = (xn - mean) / jnp.sqrt(var + 1e-5) * lng[...] + lnb[...]


def _conv_pallas(x, xj, de, ve, c, centers):
    n = x.shape[0]
    bn = _pick_block(n, [400, 200, 40, 8])
    wts = [
        c['widths2d'], centers,
        c['ee1'][0], c['ee1'][1], c['ee2'][0], c['ee2'][1],
        c['g'][0], c['g'][1],
        c['m1'][0], c['m1'][1], c['m2'][0], c['m2'][1],
        c['u1'][0], c['u1'][1], c['u2'][0], c['u2'][1],
        c['ln_g'], c['ln_b'],
    ]
    in_specs = [
        pl.BlockSpec((bn, H), lambda i: (i, 0)),
        pl.BlockSpec((bn * K, H), lambda i: (i, 0)),
        pl.BlockSpec((bn * K, 1), lambda i: (i, 0)),
        pl.BlockSpec((bn * K, 1), lambda i: (i, 0)),
    ] + [pl.BlockSpec(w.shape, lambda i: (0, 0)) for w in wts]
    return pl.pallas_call(
        _conv_body,
        grid=(n // bn,),
        in_specs=in_specs,
        out_specs=pl.BlockSpec((bn, H), lambda i: (i, 0)),
        out_shape=jax.ShapeDtypeStruct((n, H), jnp.float32),
    )(x, xj, de, ve, *wts)


def _readout_body(x2, x3, x4, b2d, ro1, bro1, ro2, bro2,
                  gn1, bgn1, gn2, bgn2, pn, bpn,
                  af_ref, ae_ref, gf_ref):
    def mm(a, b):
        return jnp.dot(a, b, preferred_element_type=jnp.float32)

    af = (x2[...] + x3[...] + x4[...]) / 3.0
    ae = mm(_silu(mm(af, ro1[...]) + bro1[...]), ro2[...]) + bro2[...]
    g1 = jnp.tanh(mm(ae, gn1[...]) + bgn1[...])
    g = mm(g1, gn2[...]) + bgn2[...]
    bvec = b2d[...]
    io64 = jax.lax.broadcasted_iota(jnp.int32, (1, B), 1)
    ohb = bvec == io64
    ohf = ohb.astype(jnp.float32)
    neg = jnp.float32(-1e38)
    gmax = jnp.max(jnp.where(ohb, g, neg), axis=0, keepdims=True)
    gmax = jnp.where(gmax < -1e37, 0.0, gmax)
    gmax_atom = jax.lax.dot_general(
        ohf, gmax, (((1,), (1,)), ((), ())),
        preferred_element_type=jnp.float32)
    ge = jnp.exp(g - gmax_atom)
    gs = jnp.sum(ohf * ge, axis=0, keepdims=True)
    gs_atom = jax.lax.dot_general(
        ohf, gs, (((1,), (1,)), ((), ())),
        preferred_element_type=jnp.float32)
    alpha = ge / gs_atom
    h = _silu(mm(ae, pn[...]) + bpn[...])
    gf = jax.lax.dot_general(
        ohf, alpha * h, (((0,), (0,)), ((), ())),
        preferred_element_type=jnp.float32)
    af_ref[...] = af
    ae_ref[...] = ae
    gf_ref[...] = gf


def _readout_pallas(x2, x3, x4, batch, params):
    n = x2.shape[0]
    b2d = batch.reshape(n, 1).astype(jnp.int32)
    wts = [
        params['ro1'][0], params['ro1'][1].reshape(1, H),
        params['ro2'][0], params['ro2'][1].reshape(1, H),
        params['gn1'][0], params['gn1'][1].reshape(1, H),
        params['gn2'][0], params['gn2'][1].reshape(1, 1),
        params['pn'][0], params['pn'][1].reshape(1, H),
    ]
    return pl.pallas_call(
        _readout_body,
        out_shape=[
            jax.ShapeDtypeStruct((n, H), jnp.float32),
            jax.ShapeDtypeStruct((n, H), jnp.float32),
            jax.ShapeDtypeStruct((B, H), jnp.float32),
        ],
    )(x2, x3, x4, b2d, *wts)


def kernel(atomic_numbers, pos, batch, params):
    n = pos.shape[0]
    pos = pos.astype(jnp.float32)
    src, dist, valid = _neighbors_pallas(pos, batch)
    x = _embed_pallas(atomic_numbers, params['embed'])
    e = n * K
    src_flat = src.reshape(e)
    de = dist.reshape(e, 1)
    ve = valid.reshape(e, 1)
    centers = jnp.linspace(0.0, CUTOFF, R).reshape(1, R).astype(jnp.float32)
    feats = [x]
    for c in params['convs']:
        cc = dict(c)
        cc['widths2d'] = params['widths'].reshape(1, R)
        cc['ee1'] = (c['ee1'][0], c['ee1'][1].reshape(1, H))
        cc['ee2'] = (c['ee2'][0], c['ee2'][1].reshape(1, H))
        cc['g'] = (c['g'][0], c['g'][1].reshape(1, H))
        cc['m1'] = (c['m1'][0], c['m1'][1].reshape(1, H))
        cc['m2'] = (c['m2'][0], c['m2'][1].reshape(1, H))
        cc['u1'] = (c['u1'][0], c['u1'][1].reshape(1, H))
        cc['u2'] = (c['u2'][0], c['u2'][1].reshape(1, H))
        cc['ln_g'] = c['ln_g'].reshape(1, H)
        cc['ln_b'] = c['ln_b'].reshape(1, H)
        nchunks = 5 if n % 2000 == 0 else 2
        cn = n // nchunks
        parts = []
        for ci in range(nchunks):
            lo, hi = ci * cn, (ci + 1) * cn
            xj_c = _gather_rows(x, src_flat[lo * K:hi * K])
            parts.append(_conv_pallas(
                x[lo:hi], xj_c, de[lo * K:hi * K], ve[lo * K:hi * K],
                cc, centers))
        x = jnp.concatenate(parts, axis=0)
        feats.append(x)
    af, ae, gf = _readout_pallas(feats[-3], feats[-2], feats[-1], batch, params)
    return af, ae, gf
